# baseline (speedup 1.0000x reference)
import functools

import jax
import jax.numpy as jnp
from jax import lax
from jax.experimental import pallas as pl
from jax.experimental.pallas import tpu as pltpu
from jax.experimental.pallas import tpu_sc as plsc

_NBUF = 4
_NSP = 3
_K = 40


def _make_gather(V, D, B):
    info = plsc.get_sparse_core_info()
    NC, NS = info.num_cores, info.num_subcores
    NW = NC * NS
    assert B % NW == 0
    b_per_w = B // NW
    assert b_per_w % 8 == 0
    K = _K
    n_chunks = (b_per_w // K) // _NBUF * _NBUF
    n_rounds = n_chunks // _NBUF
    tail = b_per_w - n_chunks * K
    assert tail % 8 == 0 and tail <= K and n_rounds >= 3
    n_sp = _NBUF - _NSP

    mesh = plsc.VectorSubcoreMesh(core_axis_name="c", subcore_axis_name="s")

    @functools.partial(
        pl.kernel,
        mesh=mesh,
        out_type=jax.ShapeDtypeStruct((B, D), jnp.float32),
        scratch_types=[
            pltpu.VMEM((b_per_w,), jnp.int32),
            pltpu.VMEM_SHARED((NS * n_sp * K, D), jnp.float32),
        ]
        + [pltpu.VMEM((K, D), jnp.float32) for _ in range(_NBUF)]
        + [pltpu.SemaphoreType.DMA for _ in range(2 * _NBUF)],
    )
    def gather_kernel(table_hbm, idx_hbm, out_hbm, idx_v, sp, *rest):
        bufs = rest[:_NBUF]
        gsems = rest[_NBUF : 2 * _NBUF]
        osems = rest[2 * _NBUF :]
        sid = lax.axis_index("s")
        wid = sid * NC + lax.axis_index("c")
        base = wid * b_per_w
        pltpu.sync_copy(idx_hbm.at[pl.ds(base, b_per_w)], idx_v)

        def slot(j, n=K):
            return sp.at[pl.ds((sid * n_sp + j - _NSP) * K, n)]

        def start_gather(c, j):
            pltpu.async_copy(
                table_hbm.at[idx_v.at[pl.ds(c * K, K)]], bufs[j], gsems[j]
            )

        def wait_gather(c, j):
            pltpu.make_async_copy(
                table_hbm.at[idx_v.at[pl.ds(c * K, K)]], bufs[j], gsems[j]
            ).wait()

        def out_src(j):
            return bufs[j] if j < _NSP else slot(j)

        def start_out(c, j):
            pltpu.async_copy(
                out_src(j), out_hbm.at[pl.ds(base + c * K, K)], osems[j]
            )

        def wait_out(c, j):
            pltpu.make_async_copy(
                out_src(j), out_hbm.at[pl.ds(base + c * K, K)], osems[j]
            ).wait()

        def step(c, j, first, issue_next):
            wait_gather(c, j)
            if not first:
                wait_out(c - _NBUF, j)
            if j >= _NSP:
                pltpu.sync_copy(bufs[j], slot(j))
            start_out(c, j)
            if issue_next:
                start_gather(c + _NBUF, j)

        for j in range(_NBUF):
            start_gather(j, j)
        for j in range(_NBUF):
            step(j, j, True, True)

        def body(i, carry):
            c0 = i * _NBUF
            for j in range(_NBUF):
                step(c0 + j, j, False, True)
            return carry

        lax.fori_loop(1, n_rounds - 1, body, 0)

        cl = (n_rounds - 1) * _NBUF
        for j in range(_NBUF):
            step(cl + j, j, False, False)
        if tail:
            toff = n_chunks * K
            tb = bufs[0].at[pl.ds(0, tail)]
            pltpu.async_copy(
                table_hbm.at[idx_v.at[pl.ds(toff, tail)]], tb, gsems[0]
            ).wait()
            wait_out(cl, 0)
            pltpu.async_copy(
                tb, out_hbm.at[pl.ds(base + toff, tail)], osems[0]
            ).wait()
            start = 1
        else:
            start = 0
        for j in range(start, _NBUF):
            wait_out(cl + j, j)

    return gather_kernel


def kernel(idx, table):
    V, D = table.shape
    orig_shape = idx.shape
    idx_flat = idx.reshape(-1).astype(jnp.int32)
    B = idx_flat.shape[0]
    out = _make_gather(V, D, B)(table, idx_flat)
    return out.reshape(*orig_shape, D)


# speedup vs baseline: 1.0086x; 1.0086x over previous
import functools

import jax
import jax.numpy as jnp
from jax import lax
from jax.experimental import pallas as pl
from jax.experimental.pallas import tpu as pltpu
from jax.experimental.pallas import tpu_sc as plsc

_NBUF = 2
_K = 56


def _make_gather(V, D, B):
    info = plsc.get_sparse_core_info()
    NC, NS = info.num_cores, info.num_subcores
    NW = NC * NS
    assert B % NW == 0
    b_per_w = B // NW
    assert b_per_w % 8 == 0
    K = _K
    n_chunks = (b_per_w // K) // _NBUF * _NBUF
    n_rounds = n_chunks // _NBUF
    tail = b_per_w - n_chunks * K
    assert tail % 8 == 0 and tail <= K and n_rounds >= 3

    mesh = plsc.VectorSubcoreMesh(core_axis_name="c", subcore_axis_name="s")

    @functools.partial(
        pl.kernel,
        mesh=mesh,
        out_type=jax.ShapeDtypeStruct((B, D), jnp.float32),
        scratch_types=[
            pltpu.VMEM((b_per_w,), jnp.int32),
            pltpu.VMEM_SHARED((NS * _NBUF * K, D), jnp.float32),
        ]
        + [pltpu.VMEM((K, D), jnp.float32) for _ in range(_NBUF)]
        + [pltpu.SemaphoreType.DMA for _ in range(2 * _NBUF)],
    )
    def gather_kernel(table_hbm, idx_hbm, out_hbm, idx_v, sp, *rest):
        bufs = rest[:_NBUF]
        gsems = rest[_NBUF : 2 * _NBUF]
        hsems = rest[2 * _NBUF :]
        sid = lax.axis_index("s")
        wid = sid * NC + lax.axis_index("c")
        base = wid * b_per_w
        pltpu.sync_copy(idx_hbm.at[pl.ds(base, b_per_w)], idx_v)

        def slot(j, n=K):
            return sp.at[pl.ds((sid * _NBUF + j) * K, n)]

        def start_gather(c, j):
            pltpu.async_copy(
                table_hbm.at[idx_v.at[pl.ds(c * K, K)]], bufs[j], gsems[j]
            )

        def wait_gather(c, j):
            pltpu.make_async_copy(
                table_hbm.at[idx_v.at[pl.ds(c * K, K)]], bufs[j], gsems[j]
            ).wait()

        def start_hbm(c, j):
            pltpu.async_copy(
                slot(j), out_hbm.at[pl.ds(base + c * K, K)], hsems[j]
            )

        def wait_hbm(c, j):
            pltpu.make_async_copy(
                slot(j), out_hbm.at[pl.ds(base + c * K, K)], hsems[j]
            ).wait()

        def step(c, j, first, issue_next):
            wait_gather(c, j)
            if not first:
                wait_hbm(c - _NBUF, j)
            pltpu.sync_copy(bufs[j], slot(j))
            start_hbm(c, j)
            if issue_next:
                start_gather(c + _NBUF, j)

        for j in range(_NBUF):
            start_gather(j, j)
        for j in range(_NBUF):
            step(j, j, True, True)

        def body(i, carry):
            c0 = i * _NBUF
            for j in range(_NBUF):
                step(c0 + j, j, False, True)
            return carry

        lax.fori_loop(1, n_rounds - 1, body, 0)

        cl = (n_rounds - 1) * _NBUF
        for j in range(_NBUF):
            step(cl + j, j, False, False)
        if tail:
            toff = n_chunks * K
            tb = bufs[0].at[pl.ds(0, tail)]
            pltpu.async_copy(
                table_hbm.at[idx_v.at[pl.ds(toff, tail)]], tb, gsems[0]
            ).wait()
            wait_hbm(cl, 0)
            pltpu.sync_copy(tb, slot(0, tail))
            pltpu.async_copy(
                slot(0, tail), out_hbm.at[pl.ds(base + toff, tail)], hsems[0]
            ).wait()
            start = 1
        else:
            start = 0
        for j in range(start, _NBUF):
            wait_hbm(cl + j, j)

    return gather_kernel


def kernel(idx, table):
    V, D = table.shape
    orig_shape = idx.shape
    idx_flat = idx.reshape(-1).astype(jnp.int32)
    B = idx_flat.shape[0]
    out = _make_gather(V, D, B)(table, idx_flat)
    return out.reshape(*orig_shape, D)
